# baseline (speedup 1.0000x reference)
import functools

import jax
import jax.numpy as jnp
from jax import lax
from jax.experimental import pallas as pl
from jax.experimental.pallas import tpu as pltpu
from jax.experimental.pallas import tpu_sc as plsc

N = 500000
NSEM = 64
NW = 32
CW = 640
NCH = 499968 // CW
TAIL0 = NCH * CW
TAIL1 = TAIL0 + 128
GRP = CW // 16

_mesh = plsc.VectorSubcoreMesh(core_axis_name="c", subcore_axis_name="s")


@functools.partial(
    pl.kernel,
    out_type=jax.ShapeDtypeStruct((NSEM, N), jnp.float32),
    mesh=_mesh,
    scratch_types=[
        pltpu.VMEM((CW,), jnp.int32),
        pltpu.VMEM((CW,), jnp.int32),
        pltpu.VMEM((CW,), jnp.int32),
        pltpu.VMEM((CW,), jnp.int32),
        pltpu.VMEM((NSEM, CW), jnp.float32),
        pltpu.VMEM((NSEM, CW), jnp.float32),
        pltpu.SemaphoreType.DMA,
        pltpu.SemaphoreType.DMA,
        pltpu.SemaphoreType.DMA,
        pltpu.SemaphoreType.DMA,
    ],
    compiler_params=pltpu.CompilerParams(
        needs_layout_passes=False, use_tc_tiling_on_sc=True
    ),
)
def _sc_onehot(
    sem_hbm, out_hbm,
    lbl_a, lbl_b, snap_a, snap_b, buf_a, buf_b,
    sem_a, sem_b, sem_la, sem_lb,
):
    wid = lax.axis_index("s") * 2 + lax.axis_index("c")
    zeros = jnp.zeros((16,), jnp.float32)
    ones = jnp.full((16,), 1.0, jnp.float32)
    lane = lax.iota(jnp.int32, 16)

    nch = jnp.where(wid < NCH % NW, NCH // NW + 1, NCH // NW)

    pltpu.async_copy(sem_hbm.at[pl.ds(wid * CW, CW)], lbl_a, sem_la)

    def zinit(i, carry):
        buf_a[i // GRP, pl.ds((i % GRP) * 16, 16)] = zeros
        buf_b[i // GRP, pl.ds((i % GRP) * 16, 16)] = zeros
        return carry

    lax.fori_loop(0, NSEM * GRP, zinit, 0)

    def process(i, lbl, lbl_nxt, snap, buf, sem, sem_l, sem_l_nxt):
        col0 = (wid + i * NW) * CW
        out_slice = out_hbm.at[:, pl.ds(col0, CW)]

        @pl.when(i >= 2)
        def _():
            pltpu.make_async_copy(buf, out_slice, sem).wait()

            def clr(g, c):
                lv = snap[pl.ds(g * 16, 16)]
                plsc.store_scatter(buf, [lv, g * 16 + lane], zeros)
                return c

            lax.fori_loop(0, GRP, clr, 0)

        pltpu.make_async_copy(sem_hbm.at[pl.ds(col0, CW)], lbl, sem_l).wait()

        @pl.when(i + 1 < nch)
        def _():
            pltpu.async_copy(
                sem_hbm.at[pl.ds(col0 + NW * CW, CW)], lbl_nxt, sem_l_nxt
            )

        def put(g, c):
            lv = lbl[pl.ds(g * 16, 16)]
            snap[pl.ds(g * 16, 16)] = lv
            plsc.store_scatter(buf, [lv, g * 16 + lane], ones)
            return c

        lax.fori_loop(0, GRP, put, 0)
        pltpu.async_copy(buf, out_slice, sem)

    def chunk_body(i, carry):
        @pl.when(i % 2 == 0)
        def _():
            process(i, lbl_a, lbl_b, snap_a, buf_a, sem_a, sem_la, sem_lb)

        @pl.when(i % 2 == 1)
        def _():
            process(i, lbl_b, lbl_a, snap_b, buf_b, sem_b, sem_lb, sem_la)

        return carry

    lax.fori_loop(0, nch, chunk_body, 0)

    pltpu.make_async_copy(buf_a, out_hbm.at[:, pl.ds(wid * CW, CW)], sem_a).wait()
    pltpu.make_async_copy(buf_b, out_hbm.at[:, pl.ds(wid * CW, CW)], sem_b).wait()


def kernel(semantic):
    out_t = _sc_onehot(semantic)
    out = out_t.T
    tail = (
        semantic[TAIL0:, None] == jnp.arange(NSEM, dtype=jnp.int32)[None, :]
    ).astype(jnp.float32)
    return lax.dynamic_update_slice(out, tail, (TAIL0, 0))


# speedup vs baseline: 1.0228x; 1.0228x over previous
import functools

import jax
import jax.numpy as jnp
from jax import lax
from jax.experimental import pallas as pl
from jax.experimental.pallas import tpu as pltpu
from jax.experimental.pallas import tpu_sc as plsc

N = 500000
NSEM = 64
NW = 32
CW = 640
NCH = 499968 // CW
TAIL0 = NCH * CW
TAIL1 = TAIL0 + 128
GRP = CW // 16

_mesh = plsc.VectorSubcoreMesh(core_axis_name="c", subcore_axis_name="s")


@functools.partial(
    pl.kernel,
    out_type=jax.ShapeDtypeStruct((NSEM, N), jnp.float32),
    mesh=_mesh,
    scratch_types=[
        pltpu.VMEM((CW,), jnp.int32),
        pltpu.VMEM((CW,), jnp.int32),
        pltpu.VMEM((CW,), jnp.int32),
        pltpu.VMEM((CW,), jnp.int32),
        pltpu.VMEM((NSEM, CW), jnp.float32),
        pltpu.VMEM((NSEM, CW), jnp.float32),
        pltpu.SemaphoreType.DMA,
        pltpu.SemaphoreType.DMA,
        pltpu.SemaphoreType.DMA,
        pltpu.SemaphoreType.DMA,
    ],
    compiler_params=pltpu.CompilerParams(
        needs_layout_passes=False, use_tc_tiling_on_sc=True
    ),
)
def _sc_onehot(
    sem_hbm, out_hbm,
    lbl_a, lbl_b, snap_a, snap_b, buf_a, buf_b,
    sem_a, sem_b, sem_la, sem_lb,
):
    wid = lax.axis_index("s") * 2 + lax.axis_index("c")
    zeros = jnp.zeros((16,), jnp.float32)
    ones = jnp.full((16,), 1.0, jnp.float32)
    lane = lax.iota(jnp.int32, 16)

    nch = jnp.where(wid < NCH % NW, NCH // NW + 1, NCH // NW)

    pltpu.async_copy(sem_hbm.at[pl.ds(wid * CW, CW)], lbl_a, sem_la)

    def zinit(i, carry):
        row = jnp.broadcast_to(i // GRP, (16,)).astype(jnp.int32)
        col = (i % GRP) * 16 + lane
        plsc.store_scatter(buf_a, [row, col], zeros)
        plsc.store_scatter(buf_b, [row, col], zeros)
        return carry

    lax.fori_loop(0, NSEM * GRP, zinit, 0)

    def process(i, lbl, lbl_nxt, snap, buf, sem, sem_l, sem_l_nxt):
        col0 = (wid + i * NW) * CW
        out_slice = out_hbm.at[:, pl.ds(col0, CW)]

        @pl.when(i >= 2)
        def _():
            pltpu.make_async_copy(buf, out_slice, sem).wait()

            def clr(g, c):
                lv = snap[pl.ds(g * 16, 16)]
                plsc.store_scatter(buf, [lv, g * 16 + lane], zeros)
                return c

            lax.fori_loop(0, GRP, clr, 0)

        pltpu.make_async_copy(sem_hbm.at[pl.ds(col0, CW)], lbl, sem_l).wait()

        @pl.when(i + 1 < nch)
        def _():
            pltpu.async_copy(
                sem_hbm.at[pl.ds(col0 + NW * CW, CW)], lbl_nxt, sem_l_nxt
            )

        def put(g, c):
            lv = lbl[pl.ds(g * 16, 16)]
            snap[pl.ds(g * 16, 16)] = lv
            plsc.store_scatter(buf, [lv, g * 16 + lane], ones)
            return c

        lax.fori_loop(0, GRP, put, 0)
        pltpu.async_copy(buf, out_slice, sem)

    def chunk_body(i, carry):
        @pl.when(i % 2 == 0)
        def _():
            process(i, lbl_a, lbl_b, snap_a, buf_a, sem_a, sem_la, sem_lb)

        @pl.when(i % 2 == 1)
        def _():
            process(i, lbl_b, lbl_a, snap_b, buf_b, sem_b, sem_lb, sem_la)

        return carry

    lax.fori_loop(0, nch, chunk_body, 0)

    pltpu.make_async_copy(buf_a, out_hbm.at[:, pl.ds(wid * CW, CW)], sem_a).wait()
    pltpu.make_async_copy(buf_b, out_hbm.at[:, pl.ds(wid * CW, CW)], sem_b).wait()


def kernel(semantic):
    out_t = _sc_onehot(semantic)
    out = out_t.T
    tail = (
        semantic[TAIL0:, None] == jnp.arange(NSEM, dtype=jnp.int32)[None, :]
    ).astype(jnp.float32)
    return lax.dynamic_update_slice(out, tail, (TAIL0, 0))
